# baseline (speedup 1.0000x reference)
import functools

import jax
import jax.numpy as jnp
from jax import lax
from jax.experimental import pallas as pl
from jax.experimental.pallas import tpu as pltpu
from jax.experimental.pallas import tpu_sc as plsc

NORM = 9.0
NC = 2
NS = 16
L = 16
NW = NC * NS

RBLK = 16
CH = 32


def _dot_group(emb_ref, koff, inp_ref, row_sel, nhid, unroll, xpose_v):
    npair = nhid // (2 * L)

    def body(i, accs):
        xs = {}
        new = []
        for j in range(L):
            r = row_sel(j)
            key = r if isinstance(r, int) else j
            if key not in xs:
                xs[key] = (inp_ref[r, pl.ds(i * 2 * L, L)],
                           inp_ref[r, pl.ds(i * 2 * L + L, L)])
            xlo, xhi = xs[key]
            e32 = emb_ref[koff + j, pl.ds(i * L, L)]
            eb = plsc.bitcast(e32, jnp.bfloat16)
            elo, ehi = plsc.unpack(eb, format=plsc.PackFormat.INTERLEAVED)
            new.append(accs[j] + elo * xlo + ehi * xhi)
        return tuple(new)

    zero = jnp.zeros((L,), jnp.float32)
    accs = plsc.parallel_loop(0, npair, 1, unroll=unroll,
                              carry=tuple(zero for _ in range(L)))(
        lambda i, accs: body(i, accs))
    for j in range(L):
        xpose_v[j, :] = accs[j]
    lane = lax.iota(jnp.int32, L)
    vec = zero
    for d in range(L):
        col = plsc.load_gather(xpose_v, [lane, lane * 0 + d])
        vec = vec + col
    return vec


def _nce_body(inp_hbm, tgt_hbm, smp_hbm, w_hbm, b_hbm,
              outt_hbm, outs_hbm,
              inp_v, tgtemb_v, smpemb_v, tgtidx_v, tgtbias_v,
              smpidx_v, smpbias_v, tscore_v, sscore_v, xpose_v,
              sem, sem0, sem1):
    batch, nhid = inp_hbm.shape
    nsample = smp_hbm.shape[0] // batch
    rpw = batch // NW
    nblk = rpw // RBLK

    wid = lax.axis_index("s") * NC + lax.axis_index("c")
    wbase = wid * rpw

    pltpu.sync_copy(tgt_hbm.at[pl.ds(wbase, rpw)], tgtidx_v)
    pltpu.async_copy(b_hbm.at[tgtidx_v], tgtbias_v, sem).wait()

    def block_body(blk, _):
        row0 = wbase + blk * RBLK
        pltpu.sync_copy(inp_hbm.at[pl.ds(row0, RBLK)], inp_v)
        pltpu.sync_copy(smp_hbm.at[pl.ds(row0 * nsample, RBLK * nsample)],
                        smpidx_v)
        pltpu.async_copy(b_hbm.at[smpidx_v], smpbias_v, sem).wait()
        pltpu.async_copy(w_hbm.at[tgtidx_v.at[pl.ds(blk * RBLK, RBLK)]],
                         tgtemb_v, sem).wait()
        tscore_v[pl.ds(blk * RBLK, RBLK)] = _dot_group(
            tgtemb_v, 0, inp_v, lambda j: j, nhid, 4, xpose_v)

        def chunk_copy(lr, c, buf, csem):
            return pltpu.make_async_copy(
                w_hbm.at[smpidx_v.at[pl.ds(lr * nsample + c * CH, CH)]],
                smpemb_v.at[buf], csem)

        chunk_copy(0, 0, 0, sem0).start()

        def row_body(lr, _):
            chunk_copy(lr, 0, 0, sem0).wait()
            chunk_copy(lr, 1, 1, sem1).start()
            for g in range(CH // L):
                vec = _dot_group(smpemb_v.at[0], g * L, inp_v,
                                 lambda j: lr, nhid, 4, xpose_v)
                sscore_v[pl.ds(lr * nsample + g * L, L)] = vec
            chunk_copy(lr, 1, 1, sem1).wait()

            @pl.when(lr < RBLK - 1)
            def _():
                chunk_copy(lr + 1, 0, 0, sem0).start()

            for g in range(CH // L):
                vec = _dot_group(smpemb_v.at[1], g * L, inp_v,
                                 lambda j: lr, nhid, 4, xpose_v)
                sscore_v[pl.ds(lr * nsample + CH + g * L, L)] = vec
            return 0

        lax.fori_loop(0, RBLK, row_body, 0)

        def post(i, _):
            v = sscore_v[pl.ds(i * L, L)] + smpbias_v[pl.ds(i * L, L)] - NORM
            sscore_v[pl.ds(i * L, L)] = jnp.exp(v)
            return 0

        lax.fori_loop(0, RBLK * nsample // L, post, 0)
        pltpu.sync_copy(sscore_v,
                        outs_hbm.at[pl.ds(row0 * nsample, RBLK * nsample)])
        return 0

    lax.fori_loop(0, nblk, block_body, 0)

    def tpost(i, _):
        v = tscore_v[pl.ds(i * L, L)] + tgtbias_v[pl.ds(i * L, L)] - NORM
        tscore_v[pl.ds(i * L, L)] = jnp.exp(v)
        return 0

    lax.fori_loop(0, rpw // L, tpost, 0)
    pltpu.sync_copy(tscore_v, outt_hbm.at[pl.ds(wbase, rpw)])


def kernel(input, target, sample, W, b):
    batch, nhid = input.shape
    nsample = sample.shape[1]
    rpw = batch // NW

    mesh = plsc.VectorSubcoreMesh(core_axis_name="c", subcore_axis_name="s",
                                  num_cores=NC, num_subcores=NS)
    run = pl.kernel(
        _nce_body,
        out_type=(
            jax.ShapeDtypeStruct((batch,), jnp.float32),
            jax.ShapeDtypeStruct((batch * nsample,), jnp.float32),
        ),
        mesh=mesh,
        compiler_params=pltpu.CompilerParams(needs_layout_passes=False),
        scratch_types=[
            pltpu.VMEM((RBLK, nhid), jnp.float32),
            pltpu.VMEM((RBLK, nhid // 2), jnp.int32),
            pltpu.VMEM((2, CH, nhid // 2), jnp.int32),
            pltpu.VMEM((rpw,), jnp.int32),
            pltpu.VMEM((rpw,), jnp.float32),
            pltpu.VMEM((RBLK * nsample,), jnp.int32),
            pltpu.VMEM((RBLK * nsample,), jnp.float32),
            pltpu.VMEM((rpw,), jnp.float32),
            pltpu.VMEM((RBLK * nsample,), jnp.float32),
            pltpu.VMEM((L, L), jnp.float32),
            pltpu.SemaphoreType.DMA,
            pltpu.SemaphoreType.DMA,
            pltpu.SemaphoreType.DMA,
        ],
    )
    wp = lax.bitcast_convert_type(
        W.astype(jnp.bfloat16).reshape(W.shape[0], nhid // 2, 2), jnp.int32)
    xd = input.reshape(batch, nhid // 32, 16, 2)
    xd = xd.transpose(0, 1, 3, 2).reshape(batch, nhid)
    out_t, out_s = run(xd, target, sample.reshape(-1), wp, b.reshape(-1))
    return (out_t, out_s.reshape(batch, nsample), sample)


# speedup vs baseline: 2.2130x; 2.2130x over previous
import functools

import jax
import jax.numpy as jnp
from jax import lax
from jax.experimental import pallas as pl
from jax.experimental.pallas import tpu as pltpu
from jax.experimental.pallas import tpu_sc as plsc

NORM = 9.0
NC = 2
NS = 16
L = 16
NW = NC * NS

RBLK = 16
CH = 32


def _dot_groups(emb_ref, kg, inp_ref, row_sel, nhid, unroll, xpose_v):
    dc = nhid // L

    def body(i, accs):
        xs = {}
        new = []
        for j in range(kg):
            r = row_sel(j)
            key = r if isinstance(r, int) else j
            if key not in xs:
                xs[key] = inp_ref[r, pl.ds(i * L, L)]
            e = emb_ref[j, pl.ds(i * L, L)]
            new.append(accs[j] + e * xs[key])
        return tuple(new)

    zero = jnp.zeros((L,), jnp.float32)
    accs = plsc.parallel_loop(0, dc, 1, unroll=unroll,
                              carry=tuple(zero for _ in range(kg)))(
        lambda i, accs: body(i, accs))
    lane = lax.iota(jnp.int32, L)
    vecs = []
    for g in range(kg // L):
        for j in range(L):
            xpose_v[j, :] = accs[g * L + j]
        vec = zero
        for d in range(L):
            col = plsc.load_gather(xpose_v, [lane, lane * 0 + d])
            vec = vec + col
        vecs.append(vec)
    return vecs


def _nce_body(inp_hbm, tgt_hbm, smp_hbm, w_hbm, b_hbm,
              outt_hbm, outs_hbm,
              inp_v, tgtemb_v, smpemb_v, tgtidx_v, tgtbias_v,
              smpidx_v, smpbias_v, tscore_v, sscore_v, xpose_v,
              sem, sem0, sem1):
    batch, nhid = inp_hbm.shape
    nsample = smp_hbm.shape[0] // batch
    rpw = batch // NW
    nblk = rpw // RBLK

    wid = lax.axis_index("s") * NC + lax.axis_index("c")
    wbase = wid * rpw

    pltpu.sync_copy(tgt_hbm.at[pl.ds(wbase, rpw)], tgtidx_v)
    pltpu.async_copy(b_hbm.at[tgtidx_v], tgtbias_v, sem).wait()

    def block_body(blk, _):
        row0 = wbase + blk * RBLK
        pltpu.sync_copy(inp_hbm.at[pl.ds(row0, RBLK)], inp_v)
        pltpu.sync_copy(smp_hbm.at[pl.ds(row0 * nsample, RBLK * nsample)],
                        smpidx_v)
        pltpu.async_copy(b_hbm.at[smpidx_v], smpbias_v, sem).wait()
        pltpu.async_copy(w_hbm.at[tgtidx_v.at[pl.ds(blk * RBLK, RBLK)]],
                         tgtemb_v, sem).wait()
        tscore_v[pl.ds(blk * RBLK, RBLK)] = _dot_groups(
            tgtemb_v, L, inp_v, lambda j: j, nhid, 4, xpose_v)[0]

        def chunk_copy(lr, c, buf, csem):
            return pltpu.make_async_copy(
                w_hbm.at[smpidx_v.at[pl.ds(lr * nsample + c * CH, CH)]],
                smpemb_v.at[buf], csem)

        chunk_copy(0, 0, 0, sem0).start()

        def row_body(lr, _):
            chunk_copy(lr, 0, 0, sem0).wait()
            chunk_copy(lr, 1, 1, sem1).start()
            vecs = _dot_groups(smpemb_v.at[0], CH, inp_v,
                               lambda j: lr, nhid, 4, xpose_v)
            for g in range(CH // L):
                sscore_v[pl.ds(lr * nsample + g * L, L)] = vecs[g]
            chunk_copy(lr, 1, 1, sem1).wait()

            @pl.when(lr < RBLK - 1)
            def _():
                chunk_copy(lr + 1, 0, 0, sem0).start()

            vecs = _dot_groups(smpemb_v.at[1], CH, inp_v,
                               lambda j: lr, nhid, 4, xpose_v)
            for g in range(CH // L):
                sscore_v[pl.ds(lr * nsample + CH + g * L, L)] = vecs[g]
            return 0

        lax.fori_loop(0, RBLK, row_body, 0)

        def post(i, _):
            v = sscore_v[pl.ds(i * L, L)] + smpbias_v[pl.ds(i * L, L)] - NORM
            sscore_v[pl.ds(i * L, L)] = jnp.exp(v)
            return 0

        lax.fori_loop(0, RBLK * nsample // L, post, 0)
        pltpu.sync_copy(sscore_v,
                        outs_hbm.at[pl.ds(row0 * nsample, RBLK * nsample)])
        return 0

    lax.fori_loop(0, nblk, block_body, 0)

    def tpost(i, _):
        v = tscore_v[pl.ds(i * L, L)] + tgtbias_v[pl.ds(i * L, L)] - NORM
        tscore_v[pl.ds(i * L, L)] = jnp.exp(v)
        return 0

    lax.fori_loop(0, rpw // L, tpost, 0)
    pltpu.sync_copy(tscore_v, outt_hbm.at[pl.ds(wbase, rpw)])


def kernel(input, target, sample, W, b):
    batch, nhid = input.shape
    nsample = sample.shape[1]
    rpw = batch // NW

    mesh = plsc.VectorSubcoreMesh(core_axis_name="c", subcore_axis_name="s",
                                  num_cores=NC, num_subcores=NS)
    run = pl.kernel(
        _nce_body,
        out_type=(
            jax.ShapeDtypeStruct((batch,), jnp.float32),
            jax.ShapeDtypeStruct((batch * nsample,), jnp.float32),
        ),
        mesh=mesh,
        compiler_params=pltpu.CompilerParams(needs_layout_passes=False),
        scratch_types=[
            pltpu.VMEM((RBLK, nhid), jnp.float32),
            pltpu.VMEM((RBLK, nhid), jnp.float32),
            pltpu.VMEM((2, CH, nhid), jnp.float32),
            pltpu.VMEM((rpw,), jnp.int32),
            pltpu.VMEM((rpw,), jnp.float32),
            pltpu.VMEM((RBLK * nsample,), jnp.int32),
            pltpu.VMEM((RBLK * nsample,), jnp.float32),
            pltpu.VMEM((rpw,), jnp.float32),
            pltpu.VMEM((RBLK * nsample,), jnp.float32),
            pltpu.VMEM((L, L), jnp.float32),
            pltpu.SemaphoreType.DMA,
            pltpu.SemaphoreType.DMA,
            pltpu.SemaphoreType.DMA,
        ],
    )
    out_t, out_s = run(input, target, sample.reshape(-1), W, b.reshape(-1))
    return (out_t, out_s.reshape(batch, nsample), sample)


# speedup vs baseline: 2.6136x; 1.1810x over previous
import functools

import jax
import jax.numpy as jnp
from jax import lax
from jax.experimental import pallas as pl
from jax.experimental.pallas import tpu as pltpu
from jax.experimental.pallas import tpu_sc as plsc

NORM = 9.0
NC = 2
NS = 16
L = 16
NW = NC * NS

RBLK = 16
CH = 32


def _dot_group(emb_ref, koff, inp_ref, row_sel, nhid, unroll, xpose_v):
    dc = nhid // L

    def body(i, accs):
        xs = {}
        new = []
        for j in range(L):
            r = row_sel(j)
            key = r if isinstance(r, int) else j
            if key not in xs:
                xs[key] = inp_ref[r, pl.ds(i * L, L)]
            e = emb_ref[koff + j, pl.ds(i * L, L)]
            new.append(accs[j] + e * xs[key])
        return tuple(new)

    zero = jnp.zeros((L,), jnp.float32)
    accs = plsc.parallel_loop(0, dc, 1, unroll=unroll,
                              carry=tuple(zero for _ in range(L)))(
        lambda i, accs: body(i, accs))
    for j in range(L):
        xpose_v[j, :] = accs[j]
    lane = lax.iota(jnp.int32, L)
    vec = zero
    for d in range(L):
        col = plsc.load_gather(xpose_v, [lane, lane * 0 + d])
        vec = vec + col
    return vec


def _nce_body(inp_hbm, tgt_hbm, smp_hbm, w_hbm, b_hbm,
              outt_hbm, outs_hbm,
              inp_v, tgtemb_v, smpemb_v, tgtidx_v, tgtbias_v,
              smpidx0_v, smpidx1_v, smpbias0_v, smpbias1_v, tscore_v, sscore_v, xpose_v,
              sem, sem0, sem1, sem_inp, sem_idx, sem_bias, sem_out):
    batch, nhid = inp_hbm.shape
    nsample = smp_hbm.shape[0] // batch
    rpw = batch // NW
    nblk = rpw // RBLK

    wid = lax.axis_index("s") * NC + lax.axis_index("c")
    wbase = wid * rpw

    pltpu.sync_copy(tgt_hbm.at[pl.ds(wbase, rpw)], tgtidx_v)
    pltpu.async_copy(b_hbm.at[tgtidx_v], tgtbias_v, sem).wait()

    def inp_copy(blk, s):
        return pltpu.make_async_copy(
            inp_hbm.at[pl.ds(wbase + blk * RBLK, RBLK)], inp_v.at[s], sem_inp)

    def idx_copy(blk, s):
        return pltpu.make_async_copy(
            smp_hbm.at[pl.ds((wbase + blk * RBLK) * nsample, RBLK * nsample)],
            smpidx0_v if s == 0 else smpidx1_v, sem_idx)

    def out_copy(blk, s):
        return pltpu.make_async_copy(
            sscore_v.at[s],
            outs_hbm.at[pl.ds((wbase + blk * RBLK) * nsample,
                              RBLK * nsample)], sem_out)

    inp_copy(0, 0).start()
    idx_copy(0, 0).start()

    def half_block(blk2, s):
        blk = 2 * blk2 + s
        sidx_v = smpidx0_v if s == 0 else smpidx1_v
        sbias_v = smpbias0_v if s == 0 else smpbias1_v
        inp_copy(blk, s).wait()
        idx_copy(blk, s).wait()
        pltpu.async_copy(b_hbm.at[sidx_v], sbias_v, sem_bias).start()

        @pl.when(blk < nblk - 1)
        def _():
            inp_copy(blk + 1, 1 - s).start()
            idx_copy(blk + 1, 1 - s).start()

        pltpu.async_copy(w_hbm.at[tgtidx_v.at[pl.ds(blk * RBLK, RBLK)]],
                         tgtemb_v, sem).wait()
        tscore_v[pl.ds(blk * RBLK, RBLK)] = _dot_group(
            tgtemb_v, 0, inp_v.at[s], lambda j: j, nhid, 4, xpose_v)

        def chunk_copy(lr, c, buf, csem):
            return pltpu.make_async_copy(
                w_hbm.at[sidx_v.at[pl.ds(lr * nsample + c * CH, CH)]],
                smpemb_v.at[buf], csem)

        chunk_copy(0, 0, 0, sem0).start()

        def row_body(lr, _):
            chunk_copy(lr, 0, 0, sem0).wait()
            chunk_copy(lr, 1, 1, sem1).start()
            for g in range(CH // L):
                vec = _dot_group(smpemb_v.at[0], g * L, inp_v.at[s],
                                 lambda j: lr, nhid, 4, xpose_v)
                sscore_v[s, pl.ds(lr * nsample + g * L, L)] = vec
            chunk_copy(lr, 1, 1, sem1).wait()

            @pl.when(lr < RBLK - 1)
            def _():
                chunk_copy(lr + 1, 0, 0, sem0).start()

            for g in range(CH // L):
                vec = _dot_group(smpemb_v.at[1], g * L, inp_v.at[s],
                                 lambda j: lr, nhid, 4, xpose_v)
                sscore_v[s, pl.ds(lr * nsample + CH + g * L, L)] = vec
            return 0

        lax.fori_loop(0, RBLK, row_body, 0)

        @pl.when(blk >= 2)
        def _():
            out_copy(blk - 2, s).wait()

        pltpu.make_async_copy(b_hbm.at[sidx_v], sbias_v, sem_bias).wait()

        def post(i, _):
            v = (sscore_v[s, pl.ds(i * L, L)]
                 + sbias_v[pl.ds(i * L, L)] - NORM)
            sscore_v[s, pl.ds(i * L, L)] = jnp.exp(v)
            return 0

        lax.fori_loop(0, RBLK * nsample // L, post, 0)
        out_copy(blk, s).start()

    def blk2_body(blk2, _):
        half_block(blk2, 0)
        half_block(blk2, 1)
        return 0

    lax.fori_loop(0, nblk // 2, blk2_body, 0)
    out_copy(nblk - 2, 0).wait()
    out_copy(nblk - 1, 1).wait()

    def tpost(i, _):
        v = tscore_v[pl.ds(i * L, L)] + tgtbias_v[pl.ds(i * L, L)] - NORM
        tscore_v[pl.ds(i * L, L)] = jnp.exp(v)
        return 0

    lax.fori_loop(0, rpw // L, tpost, 0)
    pltpu.sync_copy(tscore_v, outt_hbm.at[pl.ds(wbase, rpw)])


def kernel(input, target, sample, W, b):
    batch, nhid = input.shape
    nsample = sample.shape[1]
    rpw = batch // NW

    mesh = plsc.VectorSubcoreMesh(core_axis_name="c", subcore_axis_name="s",
                                  num_cores=NC, num_subcores=NS)
    run = pl.kernel(
        _nce_body,
        out_type=(
            jax.ShapeDtypeStruct((batch,), jnp.float32),
            jax.ShapeDtypeStruct((batch * nsample,), jnp.float32),
        ),
        mesh=mesh,
        compiler_params=pltpu.CompilerParams(needs_layout_passes=False),
        scratch_types=[
            pltpu.VMEM((2, RBLK, nhid), jnp.float32),
            pltpu.VMEM((RBLK, nhid), jnp.float32),
            pltpu.VMEM((2, CH, nhid), jnp.float32),
            pltpu.VMEM((rpw,), jnp.int32),
            pltpu.VMEM((rpw,), jnp.float32),
            pltpu.VMEM((RBLK * nsample,), jnp.int32),
            pltpu.VMEM((RBLK * nsample,), jnp.int32),
            pltpu.VMEM((RBLK * nsample,), jnp.float32),
            pltpu.VMEM((RBLK * nsample,), jnp.float32),
            pltpu.VMEM((rpw,), jnp.float32),
            pltpu.VMEM((2, RBLK * nsample), jnp.float32),
            pltpu.VMEM((L, L), jnp.float32),
            pltpu.SemaphoreType.DMA,
            pltpu.SemaphoreType.DMA,
            pltpu.SemaphoreType.DMA,
            pltpu.SemaphoreType.DMA,
            pltpu.SemaphoreType.DMA,
            pltpu.SemaphoreType.DMA,
            pltpu.SemaphoreType.DMA,
        ],
    )
    out_t, out_s = run(input, target, sample.reshape(-1), W, b.reshape(-1))
    return (out_t, out_s.reshape(batch, nsample), sample)


# speedup vs baseline: 2.6401x; 1.0101x over previous
import functools

import jax
import jax.numpy as jnp
from jax import lax
from jax.experimental import pallas as pl
from jax.experimental.pallas import tpu as pltpu
from jax.experimental.pallas import tpu_sc as plsc

NORM = 9.0
NC = 2
NS = 16
L = 16
NW = NC * NS

RBLK = 16
CH = 32


def _dot_group(emb_ref, koff, inp_ref, row_sel, nhid, unroll, xpose_v):
    dc = nhid // L

    def body(i, accs):
        xs = {}
        new = []
        for j in range(L):
            r = row_sel(j)
            key = r if isinstance(r, int) else j
            if key not in xs:
                xs[key] = inp_ref[r, pl.ds(i * L, L)]
            e = emb_ref[koff + j, pl.ds(i * L, L)]
            new.append(accs[j] + e * xs[key])
        return tuple(new)

    zero = jnp.zeros((L,), jnp.float32)
    accs = plsc.parallel_loop(0, dc, 1, unroll=unroll,
                              carry=tuple(zero for _ in range(L)))(
        lambda i, accs: body(i, accs))
    for j in range(L):
        xpose_v[j, :] = accs[j]
    lane = lax.iota(jnp.int32, L)
    vec = zero
    for d in range(L):
        col = plsc.load_gather(xpose_v, [lane, lane * 0 + d])
        vec = vec + col
    return vec


def _nce_body(inp_hbm, tgt_hbm, smp_hbm, w_hbm, b_hbm,
              outt_hbm, outs_hbm,
              inp_v, tgtemb_v, smpemb_v, tgtidx_v, tgtbias_v,
              smpidx0_v, smpidx1_v, smpbias0_v, smpbias1_v, tscore_v, sscore_v, xpose_v,
              sem, sem0, sem1, sem_inp, sem_idx, sem_bias, sem_out):
    batch, nhid = inp_hbm.shape
    nsample = smp_hbm.shape[0] // batch
    rpw = batch // NW
    nblk = rpw // RBLK

    wid = lax.axis_index("s") * NC + lax.axis_index("c")
    wbase = wid * rpw

    pltpu.sync_copy(tgt_hbm.at[pl.ds(wbase, rpw)], tgtidx_v)
    pltpu.async_copy(b_hbm.at[tgtidx_v], tgtbias_v, sem).wait()

    def inp_copy(blk, s):
        return pltpu.make_async_copy(
            inp_hbm.at[pl.ds(wbase + blk * RBLK, RBLK)], inp_v.at[s], sem_inp)

    def idx_copy(blk, s):
        return pltpu.make_async_copy(
            smp_hbm.at[pl.ds((wbase + blk * RBLK) * nsample, RBLK * nsample)],
            smpidx0_v if s == 0 else smpidx1_v, sem_idx)

    def out_copy(blk, s):
        return pltpu.make_async_copy(
            sscore_v.at[s],
            outs_hbm.at[pl.ds((wbase + blk * RBLK) * nsample,
                              RBLK * nsample)], sem_out)

    inp_copy(0, 0).start()
    idx_copy(0, 0).start()

    def half_block(blk2, s):
        blk = 2 * blk2 + s
        sidx_v = smpidx0_v if s == 0 else smpidx1_v
        sbias_v = smpbias0_v if s == 0 else smpbias1_v
        inp_copy(blk, s).wait()
        idx_copy(blk, s).wait()
        pltpu.async_copy(b_hbm.at[sidx_v], sbias_v, sem_bias).start()

        @pl.when(blk < nblk - 1)
        def _():
            inp_copy(blk + 1, 1 - s).start()
            idx_copy(blk + 1, 1 - s).start()

        tgt_gather = pltpu.make_async_copy(
            w_hbm.at[tgtidx_v.at[pl.ds(blk * RBLK, RBLK)]], tgtemb_v, sem)
        tgt_gather.start()

        def chunk_copy(lr, c, buf, csem):
            return pltpu.make_async_copy(
                w_hbm.at[sidx_v.at[pl.ds(lr * nsample + c * CH, CH)]],
                smpemb_v.at[buf], csem)

        chunk_copy(0, 0, 0, sem0).start()

        def row_body(lr, _):
            chunk_copy(lr, 0, 0, sem0).wait()
            chunk_copy(lr, 1, 1, sem1).start()
            for g in range(CH // L):
                vec = _dot_group(smpemb_v.at[0], g * L, inp_v.at[s],
                                 lambda j: lr, nhid, 4, xpose_v)
                sscore_v[s, pl.ds(lr * nsample + g * L, L)] = vec
            chunk_copy(lr, 1, 1, sem1).wait()

            @pl.when(lr < RBLK - 1)
            def _():
                chunk_copy(lr + 1, 0, 0, sem0).start()

            for g in range(CH // L):
                vec = _dot_group(smpemb_v.at[1], g * L, inp_v.at[s],
                                 lambda j: lr, nhid, 4, xpose_v)
                sscore_v[s, pl.ds(lr * nsample + CH + g * L, L)] = vec
            return 0

        lax.fori_loop(0, RBLK, row_body, 0)

        tgt_gather.wait()
        tscore_v[pl.ds(blk * RBLK, RBLK)] = _dot_group(
            tgtemb_v, 0, inp_v.at[s], lambda j: j, nhid, 4, xpose_v)

        @pl.when(blk >= 2)
        def _():
            out_copy(blk - 2, s).wait()

        pltpu.make_async_copy(b_hbm.at[sidx_v], sbias_v, sem_bias).wait()

        def post(i, _):
            v = (sscore_v[s, pl.ds(i * L, L)]
                 + sbias_v[pl.ds(i * L, L)] - NORM)
            sscore_v[s, pl.ds(i * L, L)] = jnp.exp(v)
            return 0

        lax.fori_loop(0, RBLK * nsample // L, post, 0)
        out_copy(blk, s).start()

    def blk2_body(blk2, _):
        half_block(blk2, 0)
        half_block(blk2, 1)
        return 0

    lax.fori_loop(0, nblk // 2, blk2_body, 0)
    out_copy(nblk - 2, 0).wait()
    out_copy(nblk - 1, 1).wait()

    def tpost(i, _):
        v = tscore_v[pl.ds(i * L, L)] + tgtbias_v[pl.ds(i * L, L)] - NORM
        tscore_v[pl.ds(i * L, L)] = jnp.exp(v)
        return 0

    lax.fori_loop(0, rpw // L, tpost, 0)
    pltpu.sync_copy(tscore_v, outt_hbm.at[pl.ds(wbase, rpw)])


def kernel(input, target, sample, W, b):
    batch, nhid = input.shape
    nsample = sample.shape[1]
    rpw = batch // NW

    mesh = plsc.VectorSubcoreMesh(core_axis_name="c", subcore_axis_name="s",
                                  num_cores=NC, num_subcores=NS)
    run = pl.kernel(
        _nce_body,
        out_type=(
            jax.ShapeDtypeStruct((batch,), jnp.float32),
            jax.ShapeDtypeStruct((batch * nsample,), jnp.float32),
        ),
        mesh=mesh,
        compiler_params=pltpu.CompilerParams(needs_layout_passes=False),
        scratch_types=[
            pltpu.VMEM((2, RBLK, nhid), jnp.float32),
            pltpu.VMEM((RBLK, nhid), jnp.float32),
            pltpu.VMEM((2, CH, nhid), jnp.float32),
            pltpu.VMEM((rpw,), jnp.int32),
            pltpu.VMEM((rpw,), jnp.float32),
            pltpu.VMEM((RBLK * nsample,), jnp.int32),
            pltpu.VMEM((RBLK * nsample,), jnp.int32),
            pltpu.VMEM((RBLK * nsample,), jnp.float32),
            pltpu.VMEM((RBLK * nsample,), jnp.float32),
            pltpu.VMEM((rpw,), jnp.float32),
            pltpu.VMEM((2, RBLK * nsample), jnp.float32),
            pltpu.VMEM((L, L), jnp.float32),
            pltpu.SemaphoreType.DMA,
            pltpu.SemaphoreType.DMA,
            pltpu.SemaphoreType.DMA,
            pltpu.SemaphoreType.DMA,
            pltpu.SemaphoreType.DMA,
            pltpu.SemaphoreType.DMA,
            pltpu.SemaphoreType.DMA,
        ],
    )
    out_t, out_s = run(input, target, sample.reshape(-1), W, b.reshape(-1))
    return (out_t, out_s.reshape(batch, nsample), sample)
